# baseline (speedup 1.0000x reference)
import functools
import jax
import jax.numpy as jnp
from jax import lax
from jax.experimental import pallas as pl
from jax.experimental.pallas import tpu as pltpu
from jax.experimental.pallas import tpu_sc as plsc

P = 0.15

_TC_BLOCK = 25000

_NC = 2
_NS = 16
_NW = _NC * _NS
_YCHUNK = 2000
_YT = 100000 // _YCHUNK
_YKMAX = -(-_YT // _NW)
_YTAIL = _YT - _NW * (_YKMAX - 1)


def _tc_body(mask_ref, x_ref, o_ref):
    keep = (mask_ref[...] >= P).astype(x_ref.dtype)
    o_ref[...] = x_ref[...] * keep


def _tc_mask(x, mask_u):
    n, d = x.shape
    grid = n // _TC_BLOCK
    return pl.pallas_call(
        _tc_body,
        grid=(grid,),
        in_specs=[
            pl.BlockSpec((1, d), lambda i: (0, 0)),
            pl.BlockSpec((_TC_BLOCK, d), lambda i: (i, 0)),
        ],
        out_specs=pl.BlockSpec((_TC_BLOCK, d), lambda i: (i, 0)),
        out_shape=jax.ShapeDtypeStruct((n, d), x.dtype),
    )(mask_u.reshape(1, d), x)


def _sc_passthrough(y):
    (n,) = y.shape
    mesh = plsc.VectorSubcoreMesh(core_axis_name="c", subcore_axis_name="s")

    @functools.partial(
        pl.kernel,
        out_type=jax.ShapeDtypeStruct((n,), y.dtype),
        mesh=mesh,
        scratch_types=[
            pltpu.VMEM((_YCHUNK,), jnp.int32),
        ],
    )
    def run(y_hbm, out_hbm, buf):
        wid = lax.axis_index("s") * _NC + lax.axis_index("c")

        def chunk(k):
            return pl.ds((wid + _NW * k) * _YCHUNK, _YCHUNK)

        def move(k):
            pltpu.sync_copy(y_hbm.at[chunk(k)], buf)
            pltpu.sync_copy(buf, out_hbm.at[chunk(k)])

        for k in range(_YKMAX):
            if k < _YKMAX - 1:
                move(k)
            else:
                pl.when(wid < _YTAIL)(lambda k=k: move(k))

    return run(y)


def kernel(x, y, mask_u):
    y_out = _sc_passthrough(y)
    x_masked = _tc_mask(x, mask_u)
    return (x_masked, y_out)


# speedup vs baseline: 1.4591x; 1.4591x over previous
import jax
import jax.numpy as jnp
from jax.experimental import pallas as pl

P = 0.15

_BLOCK_ROWS = 25000


def _mask_body(mask_ref, x_ref, o_ref):
    keep = (mask_ref[...] >= P).astype(x_ref.dtype)
    o_ref[...] = x_ref[...] * keep


def kernel(x, y, mask_u):
    n, d = x.shape
    grid = n // _BLOCK_ROWS
    x_masked = pl.pallas_call(
        _mask_body,
        grid=(grid,),
        in_specs=[
            pl.BlockSpec((1, d), lambda i: (0, 0)),
            pl.BlockSpec((_BLOCK_ROWS, d), lambda i: (i, 0)),
        ],
        out_specs=pl.BlockSpec((_BLOCK_ROWS, d), lambda i: (i, 0)),
        out_shape=jax.ShapeDtypeStruct((n, d), x.dtype),
    )(mask_u.reshape(1, d), x)
    return (x_masked, y)


# speedup vs baseline: 1.4700x; 1.0074x over previous
import jax
import jax.numpy as jnp
from jax.experimental import pallas as pl

P = 0.15

_BLOCK_ROWS = 25000


def _mask_body(mask_ref, x_ref, o_ref):
    masked = mask_ref[...] < P
    o_ref[...] = jnp.where(masked, jnp.zeros((), x_ref.dtype), x_ref[...])


def kernel(x, y, mask_u):
    n, d = x.shape
    grid = n // _BLOCK_ROWS
    x_masked = pl.pallas_call(
        _mask_body,
        grid=(grid,),
        in_specs=[
            pl.BlockSpec((1, d), lambda i: (0, 0)),
            pl.BlockSpec((_BLOCK_ROWS, d), lambda i: (i, 0)),
        ],
        out_specs=pl.BlockSpec((_BLOCK_ROWS, d), lambda i: (i, 0)),
        out_shape=jax.ShapeDtypeStruct((n, d), x.dtype),
    )(mask_u.reshape(1, d), x)
    return (x_masked, y)
